# Initial kernel scaffold; baseline (speedup 1.0000x reference)
import jax
import jax.numpy as jnp
from jax.experimental import pallas as pl


def kernel(x, lengths, emb_table, W, b):
    raise NotImplementedError("write your pallas kernel here")


# speedup vs baseline: 1.0669x; 1.0669x over previous
import functools

import jax
import jax.numpy as jnp
from jax import lax
from jax.experimental import pallas as pl
from jax.experimental.pallas import tpu as pltpu
from jax.experimental.pallas import tpu_sc as plsc

D = 64
H = 200
B = 4096
NCLS = 5
NW = 32
SPW = B // NW
CHUNK = 40
NCHUNK = H // CHUNK
LANES = 16
DC = D // LANES


def _sc_pool(x, lengths, emb_table):
  mesh = plsc.VectorSubcoreMesh(core_axis_name="c", subcore_axis_name="s")

  @functools.partial(
      pl.kernel,
      mesh=mesh,
      out_type=jax.ShapeDtypeStruct((B, 3 * D), jnp.float32),
      scratch_types=[
          pltpu.VMEM((SPW, H), jnp.int32),
          pltpu.VMEM((SPW + LANES,), jnp.int32),
          pltpu.VMEM((H, D), jnp.float32),
          pltpu.VMEM((SPW, 3 * D), jnp.float32),
          pltpu.SemaphoreType.DMA,
      ],
      compiler_params=pltpu.CompilerParams(use_tc_tiling_on_sc=False),
  )
  def k(x_hbm, len_hbm, tab_hbm, out_hbm, idx_v, len_v, rows_v,
        out_v, sem):
    wid = lax.axis_index("s") * 2 + lax.axis_index("c")
    base = wid * SPW
    pltpu.sync_copy(x_hbm.at[pl.ds(base, SPW)], idx_v)
    pltpu.sync_copy(len_hbm.at[pl.ds(base, SPW)], len_v.at[pl.ds(0, SPW)])

    def sample_body(s, carry):
      ln = len_v[pl.ds(s, LANES)][0]

      for c in range(NCHUNK):
        @pl.when(c * CHUNK < ln)
        def _():
          pltpu.async_copy(
              tab_hbm.at[idx_v.at[s, pl.ds(c * CHUNK, CHUNK)]],
              rows_v.at[pl.ds(c * CHUNK, CHUNK)],
              sem)
      for c in range(NCHUNK):
        @pl.when(c * CHUNK < ln)
        def _():
          pltpu.make_async_copy(
              tab_hbm.at[idx_v.at[s, pl.ds(c * CHUNK, CHUNK)]],
              rows_v.at[pl.ds(c * CHUNK, CHUNK)],
              sem).wait()

      def red(r, acc):
        mns, mxs, sms = acc
        nmn, nmx, nsm = [], [], []
        for c4 in range(DC):
          v = rows_v[r, pl.ds(c4 * LANES, LANES)]
          nmn.append(jnp.minimum(mns[c4], v))
          nmx.append(jnp.maximum(mxs[c4], v))
          nsm.append(sms[c4] + v)
        return (tuple(nmn), tuple(nmx), tuple(nsm))

      pos = jnp.full((LANES,), jnp.inf, dtype=jnp.float32)
      neg = jnp.full((LANES,), -jnp.inf, dtype=jnp.float32)
      zero = jnp.zeros((LANES,), dtype=jnp.float32)
      init = ((pos,) * DC, (neg,) * DC, (zero,) * DC)
      mns, mxs, sms = lax.fori_loop(0, ln, red, init)

      lnf = jnp.broadcast_to(ln.astype(jnp.float32), (LANES,))
      for c4 in range(DC):
        out_v[s, pl.ds(c4 * LANES, LANES)] = mns[c4]
        out_v[s, pl.ds(D + c4 * LANES, LANES)] = sms[c4] / lnf
        out_v[s, pl.ds(2 * D + c4 * LANES, LANES)] = mxs[c4]
      return carry

    lax.fori_loop(0, SPW, sample_body, 0)
    pltpu.sync_copy(out_v, out_hbm.at[pl.ds(base, SPW)])

  return k(x, lengths, emb_table)


def _tc_head(reps, W, b):
  def mm(r_ref, w_ref, b_ref, o_ref):
    o_ref[...] = (
        jnp.dot(r_ref[...], w_ref[...], preferred_element_type=jnp.float32)
        + b_ref[...])

  return pl.pallas_call(
      mm,
      out_shape=jax.ShapeDtypeStruct((B, NCLS), jnp.float32),
  )(reps, W, b.reshape(1, NCLS))


def kernel(x, lengths, emb_table, W, b):
  x = x.astype(jnp.int32)
  lengths = jnp.maximum(lengths.astype(jnp.int32), 1)
  reps = _sc_pool(x, lengths, emb_table)
  return _tc_head(reps, W, b)


# speedup vs baseline: 1.1990x; 1.1239x over previous
import functools

import jax
import jax.numpy as jnp
from jax import lax
from jax.experimental import pallas as pl
from jax.experimental.pallas import tpu as pltpu
from jax.experimental.pallas import tpu_sc as plsc

D = 64
H = 200
B = 4096
NCLS = 5
NW = 32
SPW = B // NW
CHUNK = 40
NCHUNK = H // CHUNK
LANES = 16
DC = D // LANES
UNROLL = 8


def _sc_pool(x, lengths, emb_table):
  mesh = plsc.VectorSubcoreMesh(core_axis_name="c", subcore_axis_name="s")

  @functools.partial(
      pl.kernel,
      mesh=mesh,
      out_type=jax.ShapeDtypeStruct((B, 3 * D), jnp.float32),
      scratch_types=[
          pltpu.VMEM((SPW, H), jnp.int32),
          pltpu.VMEM((SPW + LANES,), jnp.int32),
          pltpu.VMEM((H, D), jnp.float32),
          pltpu.VMEM((H, D), jnp.float32),
          pltpu.VMEM((SPW, 3 * D), jnp.float32),
          pltpu.SemaphoreType.DMA,
          pltpu.SemaphoreType.DMA,
      ],
      compiler_params=pltpu.CompilerParams(use_tc_tiling_on_sc=False),
  )
  def k(x_hbm, len_hbm, tab_hbm, out_hbm, idx_v, len_v, rows0, rows1,
        out_v, sem0, sem1):
    wid = lax.axis_index("s") * 2 + lax.axis_index("c")
    base = wid * SPW
    pltpu.sync_copy(x_hbm.at[pl.ds(base, SPW)], idx_v)
    pltpu.sync_copy(len_hbm.at[pl.ds(base, SPW)], len_v.at[pl.ds(0, SPW)])

    def get_len(s):
      return len_v[pl.ds(s, LANES)][0]

    def fire(s, buf, sem):
      ln = get_len(s)
      for c in range(NCHUNK):
        @pl.when(c * CHUNK < ln)
        def _():
          pltpu.async_copy(
              tab_hbm.at[idx_v.at[s, pl.ds(c * CHUNK, CHUNK)]],
              buf.at[pl.ds(c * CHUNK, CHUNK)],
              sem)

    def drain(s, buf, sem):
      ln = get_len(s)
      for c in range(NCHUNK):
        @pl.when(c * CHUNK < ln)
        def _():
          pltpu.make_async_copy(
              tab_hbm.at[idx_v.at[s, pl.ds(c * CHUNK, CHUNK)]],
              buf.at[pl.ds(c * CHUNK, CHUNK)],
              sem).wait()

    def reduce_store(s, buf):
      ln = get_len(s)

      def step(r, acc):
        mns, mxs, sms = acc
        nmn, nmx, nsm = list(mns), list(mxs), list(sms)
        for c4 in range(DC):
          v = buf[r, pl.ds(c4 * LANES, LANES)]
          nmn[c4] = jnp.minimum(nmn[c4], v)
          nmx[c4] = jnp.maximum(nmx[c4], v)
          nsm[c4] = nsm[c4] + v
        return (tuple(nmn), tuple(nmx), tuple(nsm))

      def step8(t, acc):
        r = t * UNROLL
        for u in range(UNROLL):
          acc = step(r + u, acc)
        return acc

      pos = jnp.full((LANES,), jnp.inf, dtype=jnp.float32)
      neg = jnp.full((LANES,), -jnp.inf, dtype=jnp.float32)
      zero = jnp.zeros((LANES,), dtype=jnp.float32)
      init = ((pos,) * DC, (neg,) * DC, (zero,) * DC)

      nfull = ln // UNROLL
      acc = lax.fori_loop(0, nfull, step8, init)
      mns, mxs, sms = lax.fori_loop(nfull * UNROLL, ln, step, acc)

      lnf = jnp.broadcast_to(ln.astype(jnp.float32), (LANES,))
      for c4 in range(DC):
        out_v[s, pl.ds(c4 * LANES, LANES)] = mns[c4]
        out_v[s, pl.ds(D + c4 * LANES, LANES)] = sms[c4] / lnf
        out_v[s, pl.ds(2 * D + c4 * LANES, LANES)] = mxs[c4]

    fire(0, rows0, sem0)

    def pair_body(t, carry):
      s0 = 2 * t
      fire(s0 + 1, rows1, sem1)
      drain(s0, rows0, sem0)
      reduce_store(s0, rows0)

      @pl.when(s0 + 2 < SPW)
      def _():
        fire(s0 + 2, rows0, sem0)

      drain(s0 + 1, rows1, sem1)
      reduce_store(s0 + 1, rows1)
      return carry

    lax.fori_loop(0, SPW // 2, pair_body, 0)
    pltpu.sync_copy(out_v, out_hbm.at[pl.ds(base, SPW)])

  return k(x, lengths, emb_table)


def _tc_head(reps, W, b):
  def mm(r_ref, w_ref, b_ref, o_ref):
    o_ref[...] = (
        jnp.dot(r_ref[...], w_ref[...], preferred_element_type=jnp.float32)
        + b_ref[...])

  return pl.pallas_call(
      mm,
      out_shape=jax.ShapeDtypeStruct((B, NCLS), jnp.float32),
  )(reps, W, b.reshape(1, NCLS))


def kernel(x, lengths, emb_table, W, b):
  x = x.astype(jnp.int32)
  lengths = jnp.maximum(lengths.astype(jnp.int32), 1)
  reps = _sc_pool(x, lengths, emb_table)
  return _tc_head(reps, W, b)


# speedup vs baseline: 1.8601x; 1.5513x over previous
import functools

import jax
import jax.numpy as jnp
from jax import lax
from jax.experimental import pallas as pl
from jax.experimental.pallas import tpu as pltpu
from jax.experimental.pallas import tpu_sc as plsc

D = 64
ROWW = 2 * D
H = 200
B = 4096
NCLS = 5
NW = 32
SPW = B // NW
CHUNK = 40
NCHUNK = H // CHUNK
LANES = 16
DC = D // LANES
UNROLL = 8

VB = 4096


def _tc_pack(emb_t):
  V = emb_t.shape[1]
  nblk = (V + VB - 1) // VB

  def tr(x_ref, o_ref):
    t = x_ref[...]
    o_ref[:, :D] = t[:, : VB // 2].T
    o_ref[:, D:] = t[:, VB // 2 :].T

  return pl.pallas_call(
      tr,
      out_shape=jax.ShapeDtypeStruct((nblk * VB // 2, ROWW), jnp.float32),
      grid=(nblk,),
      in_specs=[pl.BlockSpec((D, VB), lambda i: (0, i))],
      out_specs=pl.BlockSpec((VB // 2, ROWW), lambda i: (i, 0)),
  )(emb_t)


def _sc_pool(x, lengths, tab2):
  mesh = plsc.VectorSubcoreMesh(core_axis_name="c", subcore_axis_name="s")

  @functools.partial(
      pl.kernel,
      mesh=mesh,
      out_type=jax.ShapeDtypeStruct((B, 3 * D), jnp.float32),
      scratch_types=[
          pltpu.VMEM((SPW, H), jnp.int32),
          pltpu.VMEM((SPW + LANES,), jnp.int32),
          pltpu.VMEM((H, D), jnp.float32),
          pltpu.VMEM((H, D), jnp.float32),
          pltpu.VMEM((SPW, 3 * D), jnp.float32),
          pltpu.SemaphoreType.DMA,
          pltpu.SemaphoreType.DMA,
      ],
      compiler_params=pltpu.CompilerParams(use_tc_tiling_on_sc=False),
  )
  def k(x_hbm, len_hbm, tab_hbm, out_hbm, idx_v, len_v, rows0, rows1,
        out_v, sem0, sem1):
    wid = lax.axis_index("s") * 2 + lax.axis_index("c")
    base = wid * SPW
    pltpu.sync_copy(x_hbm.at[pl.ds(base, SPW)], idx_v)
    pltpu.sync_copy(len_hbm.at[pl.ds(base, SPW)], len_v.at[pl.ds(0, SPW)])

    def get_len(s):
      return len_v[pl.ds(s, LANES)][0]

    def fire(s, buf, sem):
      ln = get_len(s)
      for c in range(NCHUNK):
        @pl.when(c * CHUNK < ln)
        def _():
          pltpu.async_copy(
              tab_hbm.at[idx_v.at[s, pl.ds(c * CHUNK, CHUNK)]],
              buf.at[pl.ds(c * CHUNK, CHUNK)],
              sem)

    def drain(s, buf, sem):
      ln = get_len(s)
      for c in range(NCHUNK):
        @pl.when(c * CHUNK < ln)
        def _():
          pltpu.make_async_copy(
              tab_hbm.at[idx_v.at[s, pl.ds(c * CHUNK, CHUNK)]],
              buf.at[pl.ds(c * CHUNK, CHUNK)],
              sem).wait()

    def reduce_store(s, buf):
      ln = get_len(s)

      def step(r, acc):
        mns, mxs, sms = acc
        nmn, nmx, nsm = list(mns), list(mxs), list(sms)
        for c4 in range(DC):
          v = buf[r, pl.ds(c4 * LANES, LANES)]
          nmn[c4] = jnp.minimum(nmn[c4], v)
          nmx[c4] = jnp.maximum(nmx[c4], v)
          nsm[c4] = nsm[c4] + v
        return (tuple(nmn), tuple(nmx), tuple(nsm))

      def step8(t, acc):
        r = t * UNROLL
        for u in range(UNROLL):
          acc = step(r + u, acc)
        return acc

      pos = jnp.full((LANES,), jnp.inf, dtype=jnp.float32)
      neg = jnp.full((LANES,), -jnp.inf, dtype=jnp.float32)
      zero = jnp.zeros((LANES,), dtype=jnp.float32)
      init = ((pos,) * DC, (neg,) * DC, (zero,) * DC)

      nfull = ln // UNROLL
      acc = lax.fori_loop(0, nfull, step8, init)
      mns, mxs, sms = lax.fori_loop(nfull * UNROLL, ln, step, acc)

      lnf = jnp.broadcast_to(ln.astype(jnp.float32), (LANES,))
      for c4 in range(DC):
        out_v[s, pl.ds(c4 * LANES, LANES)] = mns[c4]
        out_v[s, pl.ds(D + c4 * LANES, LANES)] = sms[c4] / lnf
        out_v[s, pl.ds(2 * D + c4 * LANES, LANES)] = mxs[c4]

    fire(0, rows0, sem0)

    def pair_body(t, carry):
      s0 = 2 * t
      fire(s0 + 1, rows1, sem1)
      drain(s0, rows0, sem0)
      reduce_store(s0, rows0)

      @pl.when(s0 + 2 < SPW)
      def _():
        fire(s0 + 2, rows0, sem0)

      drain(s0 + 1, rows1, sem1)
      reduce_store(s0 + 1, rows1)
      return carry

    lax.fori_loop(0, SPW // 2, pair_body, 0)
    pltpu.sync_copy(out_v, out_hbm.at[pl.ds(base, SPW)])

  return k(x, lengths, tab2)


def _tc_head(reps, W, b):
  def mm(r_ref, w_ref, b_ref, o_ref):
    o_ref[...] = (
        jnp.dot(r_ref[...], w_ref[...], preferred_element_type=jnp.float32)
        + b_ref[...])

  return pl.pallas_call(
      mm,
      out_shape=jax.ShapeDtypeStruct((B, NCLS), jnp.float32),
  )(reps, W, b.reshape(1, NCLS))


def kernel(x, lengths, emb_table, W, b):
  xi = x.astype(jnp.int32)
  xq = 2 * ((xi // VB) * (VB // 2) + xi % (VB // 2)) + (xi % VB) // (VB // 2)
  lengths = jnp.maximum(lengths.astype(jnp.int32), 1)
  tab2 = _tc_pack(emb_table.T)
  tab3 = tab2.reshape(2 * tab2.shape[0], D)
  reps = _sc_pool(xq, lengths, tab3)
  return _tc_head(reps, W, b)


# speedup vs baseline: 2.1671x; 1.1651x over previous
import functools

import jax
import jax.numpy as jnp
from jax import lax
from jax.experimental import pallas as pl
from jax.experimental.pallas import tpu as pltpu
from jax.experimental.pallas import tpu_sc as plsc

D = 64
ROWW = 2 * D
H = 200
B = 4096
NCLS = 5
NW = 32
SPW = B // NW
CHUNK = 40
NCHUNK = H // CHUNK
LANES = 16
DC = D // LANES
UNROLL = 8

VB = 8192


def _tc_pack(emb_t):
  V = emb_t.shape[1]
  nblk = (V + VB - 1) // VB

  def tr(x_ref, o_ref):
    t = x_ref[...]
    o_ref[:, :D] = t[:, : VB // 2].T
    o_ref[:, D:] = t[:, VB // 2 :].T

  return pl.pallas_call(
      tr,
      out_shape=jax.ShapeDtypeStruct((nblk * VB // 2, ROWW), jnp.float32),
      grid=(nblk,),
      in_specs=[pl.BlockSpec((D, VB), lambda i: (0, i))],
      out_specs=pl.BlockSpec((VB // 2, ROWW), lambda i: (i, 0)),
  )(emb_t)


def _sc_pool(x, lengths, tab2):
  mesh = plsc.VectorSubcoreMesh(core_axis_name="c", subcore_axis_name="s")

  @functools.partial(
      pl.kernel,
      mesh=mesh,
      out_type=jax.ShapeDtypeStruct((B, 3 * D), jnp.float32),
      scratch_types=[
          pltpu.VMEM((SPW, H), jnp.int32),
          pltpu.VMEM((SPW + LANES,), jnp.int32),
          pltpu.VMEM((H, D), jnp.float32),
          pltpu.VMEM((H, D), jnp.float32),
          pltpu.VMEM((SPW, 3 * D), jnp.float32),
          pltpu.SemaphoreType.DMA,
          pltpu.SemaphoreType.DMA,
      ],
      compiler_params=pltpu.CompilerParams(use_tc_tiling_on_sc=False),
  )
  def k(x_hbm, len_hbm, tab_hbm, out_hbm, idx_v, len_v, rows0, rows1,
        out_v, sem0, sem1):
    wid = lax.axis_index("s") * 2 + lax.axis_index("c")
    base = wid * SPW
    pltpu.sync_copy(x_hbm.at[pl.ds(base, SPW)], idx_v)
    pltpu.sync_copy(len_hbm.at[pl.ds(base, SPW)], len_v.at[pl.ds(0, SPW)])

    def get_len(s):
      return len_v[pl.ds(s, LANES)][0]

    def fire(s, buf, sem):
      ln = get_len(s)
      for c in range(NCHUNK):
        @pl.when(c * CHUNK < ln)
        def _():
          pltpu.async_copy(
              tab_hbm.at[idx_v.at[s, pl.ds(c * CHUNK, CHUNK)]],
              buf.at[pl.ds(c * CHUNK, CHUNK)],
              sem)

    def drain(s, buf, sem):
      ln = get_len(s)
      for c in range(NCHUNK):
        @pl.when(c * CHUNK < ln)
        def _():
          pltpu.make_async_copy(
              tab_hbm.at[idx_v.at[s, pl.ds(c * CHUNK, CHUNK)]],
              buf.at[pl.ds(c * CHUNK, CHUNK)],
              sem).wait()

    def reduce_store(s, buf):
      ln = get_len(s)

      def step(r, acc):
        mns, mxs, sms = acc
        nmn, nmx, nsm = list(mns), list(mxs), list(sms)
        for c4 in range(DC):
          v = buf[r, pl.ds(c4 * LANES, LANES)]
          nmn[c4] = jnp.minimum(nmn[c4], v)
          nmx[c4] = jnp.maximum(nmx[c4], v)
          nsm[c4] = nsm[c4] + v
        return (tuple(nmn), tuple(nmx), tuple(nsm))

      def step8(t, acc):
        r = t * UNROLL
        for u in range(UNROLL):
          acc = step(r + u, acc)
        return acc

      pos = jnp.full((LANES,), jnp.inf, dtype=jnp.float32)
      neg = jnp.full((LANES,), -jnp.inf, dtype=jnp.float32)
      zero = jnp.zeros((LANES,), dtype=jnp.float32)
      init = ((pos,) * DC, (neg,) * DC, (zero,) * DC)

      nfull = ln // UNROLL
      acc = lax.fori_loop(0, nfull, step8, init)
      mns, mxs, sms = lax.fori_loop(nfull * UNROLL, ln, step, acc)

      lnf = jnp.broadcast_to(ln.astype(jnp.float32), (LANES,))
      for c4 in range(DC):
        out_v[s, pl.ds(c4 * LANES, LANES)] = mns[c4]
        out_v[s, pl.ds(D + c4 * LANES, LANES)] = sms[c4] / lnf
        out_v[s, pl.ds(2 * D + c4 * LANES, LANES)] = mxs[c4]

    fire(0, rows0, sem0)

    def pair_body(t, carry):
      s0 = 2 * t
      fire(s0 + 1, rows1, sem1)
      drain(s0, rows0, sem0)
      reduce_store(s0, rows0)

      @pl.when(s0 + 2 < SPW)
      def _():
        fire(s0 + 2, rows0, sem0)

      drain(s0 + 1, rows1, sem1)
      reduce_store(s0 + 1, rows1)
      return carry

    lax.fori_loop(0, SPW // 2, pair_body, 0)
    pltpu.sync_copy(out_v, out_hbm.at[pl.ds(base, SPW)])

  return k(x, lengths, tab2)


def _tc_head(reps, W, b):
  def mm(r_ref, w_ref, b_ref, o_ref):
    o_ref[...] = (
        jnp.dot(r_ref[...], w_ref[...], preferred_element_type=jnp.float32)
        + b_ref[...])

  return pl.pallas_call(
      mm,
      out_shape=jax.ShapeDtypeStruct((B, NCLS), jnp.float32),
  )(reps, W, b.reshape(1, NCLS))


def kernel(x, lengths, emb_table, W, b):
  xi = x.astype(jnp.int32)
  xq = 2 * ((xi // VB) * (VB // 2) + xi % (VB // 2)) + (xi % VB) // (VB // 2)
  lengths = jnp.maximum(lengths.astype(jnp.int32), 1)
  tab2 = _tc_pack(emb_table.T)
  tab3 = tab2.reshape(2 * tab2.shape[0], D)
  reps = _sc_pool(xq, lengths, tab3)
  return _tc_head(reps, W, b)


# speedup vs baseline: 2.3568x; 1.0875x over previous
import functools

import jax
import jax.numpy as jnp
from jax import lax
from jax.experimental import pallas as pl
from jax.experimental.pallas import tpu as pltpu
from jax.experimental.pallas import tpu_sc as plsc

D = 64
ROWW = 2 * D
H = 200
B = 4096
NCLS = 5
NW = 32
SPW = B // NW
CHUNK = 40
NCHUNK = H // CHUNK
LANES = 16
DC = D // LANES
UNROLL = 8

VB = 16384


def _tc_pack(emb_t):
  V = emb_t.shape[1]
  nblk = (V + VB - 1) // VB

  def tr(x_ref, o_ref):
    t = x_ref[...]
    o_ref[:, :D] = t[:, : VB // 2].T
    o_ref[:, D:] = t[:, VB // 2 :].T

  return pl.pallas_call(
      tr,
      out_shape=jax.ShapeDtypeStruct((nblk * VB // 2, ROWW), jnp.float32),
      grid=(nblk,),
      in_specs=[pl.BlockSpec((D, VB), lambda i: (0, i))],
      out_specs=pl.BlockSpec((VB // 2, ROWW), lambda i: (i, 0)),
  )(emb_t)


def _sc_pool(x, lengths, tab2):
  mesh = plsc.VectorSubcoreMesh(core_axis_name="c", subcore_axis_name="s")

  @functools.partial(
      pl.kernel,
      mesh=mesh,
      out_type=jax.ShapeDtypeStruct((B, 3 * D), jnp.float32),
      scratch_types=[
          pltpu.VMEM((SPW, H), jnp.int32),
          pltpu.VMEM((SPW + LANES,), jnp.int32),
          pltpu.VMEM((H, D), jnp.float32),
          pltpu.VMEM((H, D), jnp.float32),
          pltpu.VMEM((SPW, 3 * D), jnp.float32),
          pltpu.SemaphoreType.DMA,
          pltpu.SemaphoreType.DMA,
      ],
      compiler_params=pltpu.CompilerParams(use_tc_tiling_on_sc=False),
  )
  def k(x_hbm, len_hbm, tab_hbm, out_hbm, idx_v, len_v, rows0, rows1,
        out_v, sem0, sem1):
    wid = lax.axis_index("s") * 2 + lax.axis_index("c")
    base = wid * SPW
    pltpu.sync_copy(x_hbm.at[pl.ds(base, SPW)], idx_v)
    pltpu.sync_copy(len_hbm.at[pl.ds(base, SPW)], len_v.at[pl.ds(0, SPW)])

    def get_len(s):
      return len_v[pl.ds(s, LANES)][0]

    def fire(s, buf, sem):
      ln = get_len(s)
      for c in range(NCHUNK):
        @pl.when(c * CHUNK < ln)
        def _():
          pltpu.async_copy(
              tab_hbm.at[idx_v.at[s, pl.ds(c * CHUNK, CHUNK)]],
              buf.at[pl.ds(c * CHUNK, CHUNK)],
              sem)

    def drain(s, buf, sem):
      ln = get_len(s)
      for c in range(NCHUNK):
        @pl.when(c * CHUNK < ln)
        def _():
          pltpu.make_async_copy(
              tab_hbm.at[idx_v.at[s, pl.ds(c * CHUNK, CHUNK)]],
              buf.at[pl.ds(c * CHUNK, CHUNK)],
              sem).wait()

    def reduce_store(s, buf):
      ln = get_len(s)

      def step(r, acc):
        mns, mxs, sms = acc
        nmn, nmx, nsm = list(mns), list(mxs), list(sms)
        for c4 in range(DC):
          v = buf[r, pl.ds(c4 * LANES, LANES)]
          nmn[c4] = jnp.minimum(nmn[c4], v)
          nmx[c4] = jnp.maximum(nmx[c4], v)
          nsm[c4] = nsm[c4] + v
        return (tuple(nmn), tuple(nmx), tuple(nsm))

      def step8(t, acc):
        r = t * UNROLL
        for u in range(UNROLL):
          acc = step(r + u, acc)
        return acc

      pos = jnp.full((LANES,), jnp.inf, dtype=jnp.float32)
      neg = jnp.full((LANES,), -jnp.inf, dtype=jnp.float32)
      zero = jnp.zeros((LANES,), dtype=jnp.float32)
      init = ((pos,) * DC, (neg,) * DC, (zero,) * DC)

      nfull = ln // UNROLL
      acc = lax.fori_loop(0, nfull, step8, init)
      mns, mxs, sms = lax.fori_loop(nfull * UNROLL, ln, step, acc)

      lnf = jnp.broadcast_to(ln.astype(jnp.float32), (LANES,))
      for c4 in range(DC):
        out_v[s, pl.ds(c4 * LANES, LANES)] = mns[c4]
        out_v[s, pl.ds(D + c4 * LANES, LANES)] = sms[c4] / lnf
        out_v[s, pl.ds(2 * D + c4 * LANES, LANES)] = mxs[c4]

    fire(0, rows0, sem0)

    def pair_body(t, carry):
      s0 = 2 * t
      fire(s0 + 1, rows1, sem1)
      drain(s0, rows0, sem0)
      reduce_store(s0, rows0)

      @pl.when(s0 + 2 < SPW)
      def _():
        fire(s0 + 2, rows0, sem0)

      drain(s0 + 1, rows1, sem1)
      reduce_store(s0 + 1, rows1)
      return carry

    lax.fori_loop(0, SPW // 2, pair_body, 0)
    pltpu.sync_copy(out_v, out_hbm.at[pl.ds(base, SPW)])

  return k(x, lengths, tab2)


def _tc_head(reps, W, b):
  def mm(r_ref, w_ref, b_ref, o_ref):
    o_ref[...] = (
        jnp.dot(r_ref[...], w_ref[...], preferred_element_type=jnp.float32)
        + b_ref[...])

  return pl.pallas_call(
      mm,
      out_shape=jax.ShapeDtypeStruct((B, NCLS), jnp.float32),
  )(reps, W, b.reshape(1, NCLS))


def kernel(x, lengths, emb_table, W, b):
  xi = x.astype(jnp.int32)
  xq = 2 * ((xi // VB) * (VB // 2) + xi % (VB // 2)) + (xi % VB) // (VB // 2)
  lengths = jnp.maximum(lengths.astype(jnp.int32), 1)
  tab2 = _tc_pack(emb_table.T)
  tab3 = tab2.reshape(2 * tab2.shape[0], D)
  reps = _sc_pool(xq, lengths, tab3)
  return _tc_head(reps, W, b)


# speedup vs baseline: 2.4595x; 1.0436x over previous
import functools

import jax
import jax.numpy as jnp
from jax import lax
from jax.experimental import pallas as pl
from jax.experimental.pallas import tpu as pltpu
from jax.experimental.pallas import tpu_sc as plsc

D = 64
ROWW = 2 * D
H = 200
B = 4096
NCLS = 5
NW = 32
SPW = B // NW
CHUNK = 40
NCHUNK = H // CHUNK
LANES = 16
DC = D // LANES
UNROLL = 8

VB = 32768


def _tc_pack(emb_t):
  V = emb_t.shape[1]
  nblk = (V + VB - 1) // VB

  def tr(x_ref, o_ref):
    t = x_ref[...]
    o_ref[:, :D] = t[:, : VB // 2].T
    o_ref[:, D:] = t[:, VB // 2 :].T

  return pl.pallas_call(
      tr,
      out_shape=jax.ShapeDtypeStruct((nblk * VB // 2, ROWW), jnp.float32),
      grid=(nblk,),
      in_specs=[pl.BlockSpec((D, VB), lambda i: (0, i))],
      out_specs=pl.BlockSpec((VB // 2, ROWW), lambda i: (i, 0)),
  )(emb_t)


def _sc_pool(x, lengths, tab2):
  mesh = plsc.VectorSubcoreMesh(core_axis_name="c", subcore_axis_name="s")

  @functools.partial(
      pl.kernel,
      mesh=mesh,
      out_type=jax.ShapeDtypeStruct((B, 3 * D), jnp.float32),
      scratch_types=[
          pltpu.VMEM((SPW, H), jnp.int32),
          pltpu.VMEM((SPW + LANES,), jnp.int32),
          pltpu.VMEM((H, D), jnp.float32),
          pltpu.VMEM((H, D), jnp.float32),
          pltpu.VMEM((SPW, 3 * D), jnp.float32),
          pltpu.SemaphoreType.DMA,
          pltpu.SemaphoreType.DMA,
      ],
      compiler_params=pltpu.CompilerParams(use_tc_tiling_on_sc=False),
  )
  def k(x_hbm, len_hbm, tab_hbm, out_hbm, idx_v, len_v, rows0, rows1,
        out_v, sem0, sem1):
    wid = lax.axis_index("s") * 2 + lax.axis_index("c")
    base = wid * SPW
    pltpu.sync_copy(x_hbm.at[pl.ds(base, SPW)], idx_v)
    pltpu.sync_copy(len_hbm.at[pl.ds(base, SPW)], len_v.at[pl.ds(0, SPW)])

    def get_len(s):
      return len_v[pl.ds(s, LANES)][0]

    def fire(s, buf, sem):
      ln = get_len(s)
      for c in range(NCHUNK):
        @pl.when(c * CHUNK < ln)
        def _():
          pltpu.async_copy(
              tab_hbm.at[idx_v.at[s, pl.ds(c * CHUNK, CHUNK)]],
              buf.at[pl.ds(c * CHUNK, CHUNK)],
              sem)

    def drain(s, buf, sem):
      ln = get_len(s)
      for c in range(NCHUNK):
        @pl.when(c * CHUNK < ln)
        def _():
          pltpu.make_async_copy(
              tab_hbm.at[idx_v.at[s, pl.ds(c * CHUNK, CHUNK)]],
              buf.at[pl.ds(c * CHUNK, CHUNK)],
              sem).wait()

    def reduce_store(s, buf):
      ln = get_len(s)

      def step(r, acc):
        mns, mxs, sms = acc
        nmn, nmx, nsm = list(mns), list(mxs), list(sms)
        for c4 in range(DC):
          v = buf[r, pl.ds(c4 * LANES, LANES)]
          nmn[c4] = jnp.minimum(nmn[c4], v)
          nmx[c4] = jnp.maximum(nmx[c4], v)
          nsm[c4] = nsm[c4] + v
        return (tuple(nmn), tuple(nmx), tuple(nsm))

      def step8(t, acc):
        r = t * UNROLL
        for u in range(UNROLL):
          acc = step(r + u, acc)
        return acc

      pos = jnp.full((LANES,), jnp.inf, dtype=jnp.float32)
      neg = jnp.full((LANES,), -jnp.inf, dtype=jnp.float32)
      zero = jnp.zeros((LANES,), dtype=jnp.float32)
      init = ((pos,) * DC, (neg,) * DC, (zero,) * DC)

      nfull = ln // UNROLL
      acc = lax.fori_loop(0, nfull, step8, init)
      mns, mxs, sms = lax.fori_loop(nfull * UNROLL, ln, step, acc)

      lnf = jnp.broadcast_to(ln.astype(jnp.float32), (LANES,))
      for c4 in range(DC):
        out_v[s, pl.ds(c4 * LANES, LANES)] = mns[c4]
        out_v[s, pl.ds(D + c4 * LANES, LANES)] = sms[c4] / lnf
        out_v[s, pl.ds(2 * D + c4 * LANES, LANES)] = mxs[c4]

    fire(0, rows0, sem0)

    def pair_body(t, carry):
      s0 = 2 * t
      fire(s0 + 1, rows1, sem1)
      drain(s0, rows0, sem0)
      reduce_store(s0, rows0)

      @pl.when(s0 + 2 < SPW)
      def _():
        fire(s0 + 2, rows0, sem0)

      drain(s0 + 1, rows1, sem1)
      reduce_store(s0 + 1, rows1)
      return carry

    lax.fori_loop(0, SPW // 2, pair_body, 0)
    pltpu.sync_copy(out_v, out_hbm.at[pl.ds(base, SPW)])

  return k(x, lengths, tab2)


def _tc_head(reps, W, b):
  def mm(r_ref, w_ref, b_ref, o_ref):
    o_ref[...] = (
        jnp.dot(r_ref[...], w_ref[...], preferred_element_type=jnp.float32)
        + b_ref[...])

  return pl.pallas_call(
      mm,
      out_shape=jax.ShapeDtypeStruct((B, NCLS), jnp.float32),
  )(reps, W, b.reshape(1, NCLS))


def kernel(x, lengths, emb_table, W, b):
  xi = x.astype(jnp.int32)
  xq = 2 * ((xi // VB) * (VB // 2) + xi % (VB // 2)) + (xi % VB) // (VB // 2)
  lengths = jnp.maximum(lengths.astype(jnp.int32), 1)
  tab2 = _tc_pack(emb_table.T)
  tab3 = tab2.reshape(2 * tab2.shape[0], D)
  reps = _sc_pool(xq, lengths, tab3)
  return _tc_head(reps, W, b)
